# baseline (speedup 1.0000x reference)
import functools

import jax
import jax.numpy as jnp
from jax import lax
from jax.experimental import pallas as pl
from jax.experimental.pallas import tpu as pltpu
from jax.experimental.pallas import tpu_sc as plsc

B = 4096
S = 200
D = 64
NC = 2
NS = 16
NW = NC * NS
ROWS_PER_W = B // NW
CHUNK = 80
NCHUNK = (2 * S) // CHUNK
IDX_PER_W = ROWS_PER_W * 2 * S
UNITS_PER_W = ROWS_PER_W * NCHUNK


def _sc_pooled_features(idx_flat, table_pad):
    mesh = plsc.VectorSubcoreMesh(
        core_axis_name="c", subcore_axis_name="s", num_cores=NC, num_subcores=NS
    )

    @functools.partial(
        pl.kernel,
        out_type=jax.ShapeDtypeStruct((B, 2 * D), jnp.float32),
        mesh=mesh,
        scratch_types=[
            pltpu.VMEM((IDX_PER_W,), jnp.int32),
            pltpu.VMEM((2, CHUNK, 2 * D), jnp.float32),
            pltpu.VMEM((ROWS_PER_W, 2 * D), jnp.float32),
            pltpu.SemaphoreType.DMA,
            pltpu.SemaphoreType.DMA,
        ],
    )
    def feat_kernel(idx_hbm, table_hbm, out_hbm, idx_v, rows_v, feat_v, s0, s1):
        wid = lax.axis_index("s") * NC + lax.axis_index("c")
        base = wid * ROWS_PER_W
        sems = (s0, s1)

        pltpu.sync_copy(idx_hbm.at[pl.ds(base * 2 * S, IDX_PER_W)], idx_v)

        def fire(unit, slot):
            pltpu.async_copy(
                table_hbm.at[idx_v.at[pl.ds(unit * CHUNK, CHUNK)]],
                rows_v.at[slot],
                sems[slot],
            )

        def drain(slot):
            pltpu.make_async_copy(
                table_hbm.at[pl.ds(0, CHUNK)], rows_v.at[slot], sems[slot]
            ).wait()

        def chunk_max(slot, lo, hi, accs):
            def body(j, a):
                return tuple(
                    jnp.maximum(a[k], rows_v[slot, j, pl.ds(k * 16, 16)])
                    for k in range(4)
                )

            return lax.fori_loop(lo, hi, body, accs, unroll=2)

        neg_inf = tuple(jnp.full((16,), -jnp.inf, jnp.float32) for _ in range(4))

        fire(0, 0)

        def pair(g, _):
            row = 2 * g
            unit = NCHUNK * row
            for r in range(2):
                p_acc, h_acc = neg_inf, neg_inf
                for c in range(NCHUNK):
                    u = r * NCHUNK + c
                    slot = u % 2
                    if r == 1 and c == NCHUNK - 1:
                        @pl.when(g < ROWS_PER_W // 2 - 1)
                        def _fire_next():
                            fire(unit + u + 1, (u + 1) % 2)
                    else:
                        fire(unit + u + 1, (u + 1) % 2)
                    drain(slot)
                    if c < 2:
                        p_acc = chunk_max(slot, 0, CHUNK, p_acc)
                    elif c == 2:
                        p_acc = chunk_max(slot, 0, CHUNK // 2, p_acc)
                        h_acc = chunk_max(slot, CHUNK // 2, CHUNK, h_acc)
                    else:
                        h_acc = chunk_max(slot, 0, CHUNK, h_acc)
                for k in range(4):
                    feat_v[row + r, pl.ds(k * 16, 16)] = p_acc[k]
                    feat_v[row + r, pl.ds(D + k * 16, 16)] = h_acc[k]
            return _

        lax.fori_loop(0, ROWS_PER_W // 2, pair, None)
        pltpu.sync_copy(feat_v, out_hbm.at[pl.ds(base, ROWS_PER_W)])

    return feat_kernel(idx_flat, table_pad)


def _tc_head(feat, W, b):
    def head_kernel(x_ref, w_ref, b_ref, o_ref):
        z = jnp.sum(x_ref[...] * w_ref[...], axis=1, keepdims=True)
        o_ref[...] = jax.nn.sigmoid(z + b_ref[0])

    return pl.pallas_call(
        head_kernel,
        in_specs=[
            pl.BlockSpec(memory_space=pltpu.VMEM),
            pl.BlockSpec(memory_space=pltpu.VMEM),
            pl.BlockSpec(memory_space=pltpu.SMEM),
        ],
        out_shape=jax.ShapeDtypeStruct((B, 1), jnp.float32),
    )(feat, W, b)


def kernel(premise, hypothesis, emb_table, W, b):
    idx_flat = jnp.concatenate(
        [premise.astype(jnp.int32), hypothesis.astype(jnp.int32)], axis=1
    ).reshape(B * 2 * S)
    table_pad = jnp.pad(emb_table, ((0, 0), (0, D)))
    feat = _sc_pooled_features(idx_flat, table_pad)
    return jnp.ravel(_tc_head(feat, W, b))
